# baseline (speedup 1.0000x reference)
import jax
import jax.numpy as jnp
from jax import lax
from jax.experimental import pallas as pl

B = 64
EMB = 256
DIM = 1024
K = 8192
F32 = jnp.float32


def _proj_body(xf_ref, xl_ref, w_ref, b_ref, of_ref, ol_ref):
    w = w_ref[...]
    b = b_ref[...]
    xf = xf_ref[...].reshape(1024, DIM)
    xl = xl_ref[...].reshape(1024, DIM)
    of_ref[...] = (jnp.dot(xf, w, preferred_element_type=F32) + b).reshape(16, 64, EMB)
    ol_ref[...] = (jnp.dot(xl, w, preferred_element_type=F32) + b).reshape(16, 64, EMB)


def _main_body(xc_ref, w1_ref, c1b_ref, w2_ref, c2b_ref, cb_ref, rv_ref,
               wout_ref, bout_ref, out_ref, p_ref):
    y = jnp.dot(xc_ref[...], w1_ref[...], preferred_element_type=F32) + c1b_ref[...]
    y = jnp.maximum(y, 0.0)
    y2 = y.reshape(2 * B, 16 * EMB)
    e = jnp.dot(y2, w2_ref[...], preferred_element_type=F32) + c2b_ref[...]
    z = e[B:] - e[:B]

    cb = cb_ref[...]
    zc = lax.dot_general(z, cb, (((1,), (1,)), ((), ())),
                         precision=lax.Precision.HIGHEST,
                         preferred_element_type=F32)
    cc = lax.dot_general(jnp.ones((1, EMB), F32), cb * cb,
                         (((1,), (1,)), ((), ())),
                         precision=lax.Precision.HIGHEST,
                         preferred_element_type=F32)
    d = cc - 2.0 * zc
    dmin = jnp.min(d, axis=1, keepdims=True)

    zz = jnp.sum(z * z, axis=1, keepdims=True)
    nq = jnp.sqrt(jnp.maximum(dmin + zz, 0.0))
    rv = rv_ref[...]
    nr = jnp.sqrt(jnp.sum(rv * rv, axis=1, keepdims=True))
    q = z + (nq / (nr + 1e-12)) * rv
    out_ref[...] = jnp.dot(q, wout_ref[...], preferred_element_type=F32) + bout_ref[...]

    col = lax.broadcasted_iota(jnp.int32, (B, K), 1)
    idx = jnp.min(jnp.where(d <= dmin, col, jnp.int32(2 ** 30)),
                  axis=1, keepdims=True)
    m = (col == idx).astype(F32)
    colcnt = lax.dot_general(jnp.ones((1, B), F32), m,
                             (((1,), (0,)), ((), ())),
                             preferred_element_type=F32)
    cnt = lax.dot_general(m, colcnt, (((1,), (1,)), ((), ())),
                          preferred_element_type=F32)
    h = -jnp.sum(jnp.log(cnt * (1.0 / B) + 1e-10)) * (1.0 / B)
    p_ref[...] = jnp.exp(h).reshape(1, 1)


def kernel(input_data_first, input_data_last, codebooks, Win, b_in, Wout, b_out, c1w, c1b, c2w, c2b):
    grid = 4
    projf, projl = pl.pallas_call(
        _proj_body,
        grid=(grid,),
        in_specs=[
            pl.BlockSpec((B // grid, 64, DIM), lambda s: (s, 0, 0)),
            pl.BlockSpec((B // grid, 64, DIM), lambda s: (s, 0, 0)),
            pl.BlockSpec((DIM, EMB), lambda s: (0, 0)),
            pl.BlockSpec((1, EMB), lambda s: (0, 0)),
        ],
        out_specs=[
            pl.BlockSpec((B // grid, 64, EMB), lambda s: (s, 0, 0)),
            pl.BlockSpec((B // grid, 64, EMB), lambda s: (s, 0, 0)),
        ],
        out_shape=[
            jax.ShapeDtypeStruct((B, 64, EMB), F32),
            jax.ShapeDtypeStruct((B, 64, EMB), F32),
        ],
    )(input_data_first, input_data_last, Win, b_in.reshape(1, EMB))

    p4 = jnp.concatenate([projf, projl], axis=0).reshape(2 * B, 8, 8, EMB)
    pp = jnp.pad(p4, ((0, 0), (1, 1), (1, 1), (0, 0)))
    taps = [pp[:, kh:kh + 8:2, kw:kw + 8:2, :] for kh in range(3) for kw in range(3)]
    xcol = jnp.concatenate(taps, axis=-1).reshape(2 * B * 16, 9 * EMB)

    w1 = jnp.transpose(c1w, (2, 3, 1, 0)).reshape(9 * EMB, EMB)
    w2 = jnp.transpose(c2w, (2, 3, 1, 0)).reshape(16 * EMB, EMB)
    rv = jax.random.normal(jax.random.key(42), (B, EMB), dtype=F32)

    out, p = pl.pallas_call(
        _main_body,
        out_shape=[
            jax.ShapeDtypeStruct((B, DIM), F32),
            jax.ShapeDtypeStruct((1, 1), F32),
        ],
    )(xcol, w1, c1b.reshape(1, EMB), w2, c2b.reshape(1, EMB), codebooks, rv,
      Wout, b_out.reshape(1, DIM))

    return out.reshape(B, 1, DIM), p.reshape(())


# speedup vs baseline: 28.1077x; 28.1077x over previous
import jax
import jax.numpy as jnp
from jax import lax
from jax.experimental import pallas as pl
from jax.experimental.pallas import tpu as pltpu

B = 64
EMB = 256
DIM = 1024
K = 8192
F32 = jnp.float32


def _body(*refs):
    xf = refs[0:8]
    xl = refs[8:16]
    (win_ref, bin_ref, w1_ref, c1b_ref, w2_ref, c2b_ref, cb_ref, rv_ref,
     wout_ref, bout_ref, out_ref, p_ref, proj_s) = refs[16:]

    r = pl.program_id(0)
    win = win_ref[...]
    bin_ = bin_ref[...]
    for c in range(8):
        gs = (2 * (r % 2) + (c % 2)) * 16 + (r // 2) * 4 + (c // 2)
        x = jnp.concatenate(
            [xf[c][...].reshape(B, DIM), xl[c][...].reshape(B, DIM)], axis=0)
        proj_s[gs] = jnp.dot(x, win, preferred_element_type=F32) + bin_

    @pl.when(r == 7)
    def _final():
        p = proj_s[...]
        groups = [p[g * 16:(g + 1) * 16] for g in range(4)]

        z4 = jnp.zeros((4, 2 * B, EMB), F32)
        z1 = jnp.zeros((1, 2 * B, EMB), F32)

        def shift_i(t):
            return jnp.concatenate([z4, t[0:12]], axis=0)

        def shift_j(t):
            return jnp.concatenate(
                [z1, t[0:3], z1, t[4:7], z1, t[8:11], z1, t[12:15]], axis=0)

        acc = c1b_ref[...].astype(F32)
        for kh in range(3):
            for kw in range(3):
                a = 0 if kh == 1 else 1
                b = 0 if kw == 1 else 1
                t = groups[a * 2 + b]
                if kw == 0:
                    t = shift_j(t)
                if kh == 0:
                    t = shift_i(t)
                acc = acc + lax.dot_general(
                    t.reshape(16 * 2 * B, EMB), w1_ref[kh * 3 + kw],
                    (((1,), (1,)), ((), ())), preferred_element_type=F32)
        y = jnp.maximum(acc, 0.0)

        y3 = y.reshape(16, 2 * B, EMB)
        e = c2b_ref[...].astype(F32)
        for t2 in range(16):
            e = e + lax.dot_general(y3[t2], w2_ref[t2],
                                    (((1,), (1,)), ((), ())),
                                    preferred_element_type=F32)
        z = e[B:] - e[:B]

        cb = cb_ref[...]
        zc = lax.dot_general(z, cb, (((1,), (1,)), ((), ())),
                             preferred_element_type=F32)
        cb2 = cb * cb
        hi = cb2.astype(jnp.bfloat16).astype(F32)
        lo = cb2 - hi
        ones_row = jnp.ones((1, EMB), F32)
        cc = (lax.dot_general(ones_row, hi, (((1,), (1,)), ((), ())),
                              preferred_element_type=F32)
              + lax.dot_general(ones_row, lo, (((1,), (1,)), ((), ())),
                                preferred_element_type=F32))
        d = cc - 2.0 * zc
        dmin = jnp.min(d, axis=1, keepdims=True)

        zz = jnp.sum(z * z, axis=1, keepdims=True)
        nq = jnp.sqrt(jnp.maximum(dmin + zz, 0.0))
        rv = rv_ref[...]
        nr = jnp.sqrt(jnp.sum(rv * rv, axis=1, keepdims=True))
        q = z + (nq / (nr + 1e-12)) * rv
        out_ref[...] = (jnp.dot(q, wout_ref[...], preferred_element_type=F32)
                        + bout_ref[...])

        col = lax.broadcasted_iota(jnp.int32, (B, K), 1)
        idx = jnp.min(jnp.where(d <= dmin, col, jnp.int32(2 ** 30)),
                      axis=1, keepdims=True)
        m = (col == idx).astype(F32)
        colcnt = lax.dot_general(jnp.ones((1, B), F32), m,
                                 (((1,), (0,)), ((), ())),
                                 preferred_element_type=F32)
        cnt = lax.dot_general(m, colcnt, (((1,), (1,)), ((), ())),
                              preferred_element_type=F32)
        h = -jnp.sum(jnp.log(cnt * (1.0 / B) + 1e-10)) * (1.0 / B)
        p_ref[...] = jnp.exp(h).reshape(1, 1)


def kernel(input_data_first, input_data_last, codebooks, Win, b_in, Wout, b_out, c1w, c1b, c2w, c2b):
    xf = input_data_first.reshape(B, 64, 1, DIM)
    xl = input_data_last.reshape(B, 64, 1, DIM)
    w1 = jnp.transpose(c1w.reshape(EMB, EMB, 9), (2, 0, 1))
    w2 = jnp.transpose(c2w.reshape(EMB, EMB, 16), (2, 0, 1))
    rv = jax.random.normal(jax.random.key(42), (B, EMB), dtype=F32)

    def col_spec(k):
        return pl.BlockSpec((B, 1, 1, DIM), lambda r, _k=k: (0, 8 * r + _k, 0, 0))

    const = lambda shape: pl.BlockSpec(shape, lambda r: (0,) * len(shape))
    out, p = pl.pallas_call(
        _body,
        grid=(8,),
        in_specs=(
            [col_spec(k) for k in range(8)]
            + [col_spec(k) for k in range(8)]
            + [
                const((DIM, EMB)),
                const((1, EMB)),
                const((9, EMB, EMB)),
                const((1, EMB)),
                const((16, EMB, EMB)),
                const((1, EMB)),
                const((K, EMB)),
                const((B, EMB)),
                const((EMB, DIM)),
                const((1, DIM)),
            ]
        ),
        out_specs=[
            const((B, DIM)),
            const((1, 1)),
        ],
        out_shape=[
            jax.ShapeDtypeStruct((B, DIM), F32),
            jax.ShapeDtypeStruct((1, 1), F32),
        ],
        scratch_shapes=[pltpu.VMEM((64, 2 * B, EMB), F32)],
    )(*([xf] * 8 + [xl] * 8),
      Win, b_in.reshape(1, EMB), w1, c1b.reshape(1, EMB), w2,
      c2b.reshape(1, EMB), codebooks, rv, Wout, b_out.reshape(1, DIM))

    return out.reshape(B, 1, DIM), p.reshape(())


# speedup vs baseline: 46.9345x; 1.6698x over previous
import jax
import jax.numpy as jnp
from jax import lax
from jax.experimental import pallas as pl
from jax.experimental.pallas import tpu as pltpu

B = 64
EMB = 256
DIM = 1024
K = 8192
F32 = jnp.float32


def _row_copies(xf_hbm, xl_hbm, xbuf, sem, r, slot):
    cps = []
    for c in range(8):
        p = 8 * r + c
        cps.append(pltpu.make_async_copy(
            xf_hbm.at[:, p, :], xbuf.at[slot, c], sem.at[slot]))
        cps.append(pltpu.make_async_copy(
            xl_hbm.at[:, p, :], xbuf.at[slot, 8 + c], sem.at[slot]))
    return cps


def _body(xf_hbm, xl_hbm, win_ref, bin_ref, w1_ref, c1b_ref, w2_ref, c2b_ref,
          cb_ref, rv_ref, wout_ref, bout_ref, out_ref, p_ref,
          proj_s, xbuf, sem):
    r = pl.program_id(0)

    @pl.when(r == 0)
    def _prime():
        for cp in _row_copies(xf_hbm, xl_hbm, xbuf, sem, r, 0):
            cp.start()

    @pl.when(r < 7)
    def _prefetch():
        for cp in _row_copies(xf_hbm, xl_hbm, xbuf, sem, r + 1, (r + 1) % 2):
            cp.start()

    slot = r % 2
    for cp in _row_copies(xf_hbm, xl_hbm, xbuf, sem, r, slot):
        cp.wait()

    win = win_ref[...]
    bin_ = bin_ref[...]
    for c in range(8):
        gs = (2 * (r % 2) + (c % 2)) * 16 + (r // 2) * 4 + (c // 2)
        x = jnp.concatenate([xbuf[slot, c], xbuf[slot, 8 + c]], axis=0)
        proj_s[gs] = jnp.dot(x, win, preferred_element_type=F32) + bin_

    @pl.when(r == 7)
    def _final():
        p = proj_s[...]
        groups = [p[g * 16:(g + 1) * 16] for g in range(4)]

        z4 = jnp.zeros((4, 2 * B, EMB), F32)
        z1 = jnp.zeros((1, 2 * B, EMB), F32)

        def shift_i(t):
            return jnp.concatenate([z4, t[0:12]], axis=0)

        def shift_j(t):
            return jnp.concatenate(
                [z1, t[0:3], z1, t[4:7], z1, t[8:11], z1, t[12:15]], axis=0)

        acc = c1b_ref[...].astype(F32)
        for kh in range(3):
            for kw in range(3):
                a = 0 if kh == 1 else 1
                b = 0 if kw == 1 else 1
                t = groups[a * 2 + b]
                if kw == 0:
                    t = shift_j(t)
                if kh == 0:
                    t = shift_i(t)
                acc = acc + lax.dot_general(
                    t.reshape(16 * 2 * B, EMB), w1_ref[kh * 3 + kw],
                    (((1,), (1,)), ((), ())), preferred_element_type=F32)
        y = jnp.maximum(acc, 0.0)

        y3 = y.reshape(16, 2 * B, EMB)
        e = c2b_ref[...].astype(F32)
        for t2 in range(16):
            e = e + lax.dot_general(y3[t2], w2_ref[t2],
                                    (((1,), (1,)), ((), ())),
                                    preferred_element_type=F32)
        z = e[B:] - e[:B]

        cb = cb_ref[...]
        zc = lax.dot_general(z, cb, (((1,), (1,)), ((), ())),
                             preferred_element_type=F32)
        cb2 = cb * cb
        hi = cb2.astype(jnp.bfloat16).astype(F32)
        lo = cb2 - hi
        ones_row = jnp.ones((1, EMB), F32)
        cc = (lax.dot_general(ones_row, hi, (((1,), (1,)), ((), ())),
                              preferred_element_type=F32)
              + lax.dot_general(ones_row, lo, (((1,), (1,)), ((), ())),
                                preferred_element_type=F32))
        d = cc - 2.0 * zc
        dmin = jnp.min(d, axis=1, keepdims=True)

        zz = jnp.sum(z * z, axis=1, keepdims=True)
        nq = jnp.sqrt(jnp.maximum(dmin + zz, 0.0))
        rv = rv_ref[...]
        nr = jnp.sqrt(jnp.sum(rv * rv, axis=1, keepdims=True))
        q = z + (nq / (nr + 1e-12)) * rv
        out_ref[...] = (jnp.dot(q, wout_ref[...], preferred_element_type=F32)
                        + bout_ref[...])

        col = lax.broadcasted_iota(jnp.int32, (B, K), 1)
        idx = jnp.min(jnp.where(d <= dmin, col, jnp.int32(2 ** 30)),
                      axis=1, keepdims=True)
        m = (col == idx).astype(F32)
        colcnt = lax.dot_general(jnp.ones((1, B), F32), m,
                                 (((1,), (0,)), ((), ())),
                                 preferred_element_type=F32)
        cnt = lax.dot_general(m, colcnt, (((1,), (1,)), ((), ())),
                              preferred_element_type=F32)
        h = -jnp.sum(jnp.log(cnt * (1.0 / B) + 1e-10)) * (1.0 / B)
        p_ref[...] = jnp.exp(h).reshape(1, 1)


def kernel(input_data_first, input_data_last, codebooks, Win, b_in, Wout, b_out, c1w, c1b, c2w, c2b):
    w1 = jnp.transpose(c1w.reshape(EMB, EMB, 9), (2, 0, 1))
    w2 = jnp.transpose(c2w.reshape(EMB, EMB, 16), (2, 0, 1))
    rv = jax.random.normal(jax.random.key(42), (B, EMB), dtype=F32)

    const = lambda shape: pl.BlockSpec(shape, lambda r: (0,) * len(shape))
    out, p = pl.pallas_call(
        _body,
        grid=(8,),
        in_specs=[
            pl.BlockSpec(memory_space=pl.ANY),
            pl.BlockSpec(memory_space=pl.ANY),
            const((DIM, EMB)),
            const((1, EMB)),
            const((9, EMB, EMB)),
            const((1, EMB)),
            const((16, EMB, EMB)),
            const((1, EMB)),
            const((K, EMB)),
            const((B, EMB)),
            const((EMB, DIM)),
            const((1, DIM)),
        ],
        out_specs=[
            const((B, DIM)),
            const((1, 1)),
        ],
        out_shape=[
            jax.ShapeDtypeStruct((B, DIM), F32),
            jax.ShapeDtypeStruct((1, 1), F32),
        ],
        scratch_shapes=[
            pltpu.VMEM((64, 2 * B, EMB), F32),
            pltpu.VMEM((2, 16, B, DIM), F32),
            pltpu.SemaphoreType.DMA((2,)),
        ],
    )(input_data_first, input_data_last,
      Win, b_in.reshape(1, EMB), w1, c1b.reshape(1, EMB), w2,
      c2b.reshape(1, EMB), codebooks, rv, Wout, b_out.reshape(1, DIM))

    return out.reshape(B, 1, DIM), p.reshape(())


# speedup vs baseline: 53.4161x; 1.1381x over previous
import jax
import jax.numpy as jnp
from jax import lax
from jax.experimental import pallas as pl
from jax.experimental.pallas import tpu as pltpu

B = 64
EMB = 256
DIM = 1024
K = 8192
F32 = jnp.float32


def _row_copies(xf_hbm, xl_hbm, xbuf, sem, r, slot):
    cps = []
    for c in range(8):
        p = 8 * r + c
        cps.append(pltpu.make_async_copy(
            xf_hbm.at[:, p, :], xbuf.at[slot, c], sem.at[slot]))
        cps.append(pltpu.make_async_copy(
            xl_hbm.at[:, p, :], xbuf.at[slot, 8 + c], sem.at[slot]))
    return cps


def _body(xf_hbm, xl_hbm, win_ref, bin_ref, w1_ref, c1b_ref, w2_ref, c2b_ref,
          cb_hbm, rv_ref, wout_hbm, bout_ref, out_ref, p_ref,
          proj_s, xbuf, e_s, cb_s, wout_s, sem, wsem):
    r = pl.program_id(0)
    cb_cp = pltpu.make_async_copy(cb_hbm, cb_s, wsem.at[0])
    wout_cp = pltpu.make_async_copy(wout_hbm, wout_s, wsem.at[1])

    @pl.when(r == 0)
    def _prime():
        cb_cp.start()
        wout_cp.start()
        for cp in _row_copies(xf_hbm, xl_hbm, xbuf, sem, r, 0):
            cp.start()

    @pl.when(r < 7)
    def _prefetch():
        for cp in _row_copies(xf_hbm, xl_hbm, xbuf, sem, r + 1, (r + 1) % 2):
            cp.start()

    slot = r % 2
    for cp in _row_copies(xf_hbm, xl_hbm, xbuf, sem, r, slot):
        cp.wait()

    win = win_ref[...]
    bin_ = bin_ref[...]
    for c in range(8):
        gs = (2 * (r % 2) + (c % 2)) * 16 + (r // 2) * 4 + (c // 2)
        x = jnp.concatenate([xbuf[slot, c], xbuf[slot, 8 + c]], axis=0)
        proj_s[gs] = jnp.dot(x, win, preferred_element_type=F32) + bin_

    @pl.when(r % 2 == 1)
    def _conv_row():
        i = (r - 1) // 2
        z1 = jnp.zeros((1, 2 * B, EMB), F32)

        def shift_j(t):
            return jnp.concatenate([z1, t[0:3]], axis=0)

        acc = c1b_ref[...].astype(F32)
        for kh in range(3):
            for kw in range(3):
                a = 0 if kh == 1 else 1
                b = 0 if kw == 1 else 1
                g = a * 2 + b
                if kh == 0:
                    base = g * 16 + (i - 1) * 4
                    t = jnp.where(i > 0,
                                  proj_s[pl.ds(jnp.maximum(base, 0), 4)], 0.0)
                else:
                    t = proj_s[pl.ds(g * 16 + i * 4, 4)]
                if kw == 0:
                    t = shift_j(t)
                acc = acc + lax.dot_general(
                    t.reshape(4 * 2 * B, EMB), w1_ref[kh * 3 + kw],
                    (((1,), (1,)), ((), ())), preferred_element_type=F32)
        y = jnp.maximum(acc, 0.0)
        y3 = y.reshape(4, 2 * B, EMB)
        part = c2b_ref[...].astype(F32) * 0.25
        for j in range(4):
            part = part + lax.dot_general(
                y3[j], w2_ref[i, j], (((1,), (1,)), ((), ())),
                preferred_element_type=F32)

        @pl.when(r == 1)
        def _init():
            e_s[...] = part

        @pl.when(r > 1)
        def _accum():
            e_s[...] = e_s[...] + part

    @pl.when(r == 7)
    def _final():
        e = e_s[...]
        z = e[B:] - e[:B]

        cb_cp.wait()
        cb = cb_s[...]
        zc = lax.dot_general(z, cb, (((1,), (1,)), ((), ())),
                             preferred_element_type=F32)
        cb2 = cb * cb
        hi = cb2.astype(jnp.bfloat16).astype(F32)
        lo = cb2 - hi
        ones_row = jnp.ones((1, EMB), F32)
        cc = (lax.dot_general(ones_row, hi, (((1,), (1,)), ((), ())),
                              preferred_element_type=F32)
              + lax.dot_general(ones_row, lo, (((1,), (1,)), ((), ())),
                                preferred_element_type=F32))
        d = cc - 2.0 * zc
        dmin = jnp.min(d, axis=1, keepdims=True)

        zz = jnp.sum(z * z, axis=1, keepdims=True)
        nq = jnp.sqrt(jnp.maximum(dmin + zz, 0.0))
        rv = rv_ref[...]
        nr = jnp.sqrt(jnp.sum(rv * rv, axis=1, keepdims=True))
        q = z + (nq / (nr + 1e-12)) * rv
        wout_cp.wait()
        out = jnp.dot(q, wout_s[...], preferred_element_type=F32) + bout_ref[...]
        out_ref[...] = out.reshape(B, 1, DIM)

        col = lax.broadcasted_iota(jnp.int32, (B, K), 1)
        idx = jnp.min(jnp.where(d <= dmin, col, jnp.int32(2 ** 30)),
                      axis=1, keepdims=True)
        m = (col == idx).astype(F32)
        colcnt = lax.dot_general(jnp.ones((1, B), F32), m,
                                 (((1,), (0,)), ((), ())),
                                 preferred_element_type=F32)
        cnt = lax.dot_general(m, colcnt, (((1,), (1,)), ((), ())),
                              preferred_element_type=F32)
        h = -jnp.sum(jnp.log(cnt * (1.0 / B) + 1e-10)) * (1.0 / B)
        p_ref[...] = jnp.exp(h).reshape(1, 1)


def kernel(input_data_first, input_data_last, codebooks, Win, b_in, Wout, b_out, c1w, c1b, c2w, c2b):
    w1 = jnp.transpose(c1w.reshape(EMB, EMB, 9), (2, 0, 1))
    w2 = jnp.transpose(c2w.reshape(EMB, EMB, 16), (2, 0, 1)).reshape(4, 4, EMB, EMB)
    with jax.ensure_compile_time_eval():
        rv = jax.random.normal(jax.random.key(42), (B, EMB), dtype=F32)

    const = lambda shape: pl.BlockSpec(shape, lambda r: (0,) * len(shape))
    out, p = pl.pallas_call(
        _body,
        grid=(8,),
        in_specs=[
            pl.BlockSpec(memory_space=pl.ANY),
            pl.BlockSpec(memory_space=pl.ANY),
            const((DIM, EMB)),
            const((1, EMB)),
            const((9, EMB, EMB)),
            const((1, EMB)),
            const((4, 4, EMB, EMB)),
            const((1, EMB)),
            pl.BlockSpec(memory_space=pl.ANY),
            const((B, EMB)),
            pl.BlockSpec(memory_space=pl.ANY),
            const((1, DIM)),
        ],
        out_specs=[
            const((B, 1, DIM)),
            const((1, 1)),
        ],
        out_shape=[
            jax.ShapeDtypeStruct((B, 1, DIM), F32),
            jax.ShapeDtypeStruct((1, 1), F32),
        ],
        scratch_shapes=[
            pltpu.VMEM((64, 2 * B, EMB), F32),
            pltpu.VMEM((2, 16, B, DIM), F32),
            pltpu.VMEM((2 * B, EMB), F32),
            pltpu.VMEM((K, EMB), F32),
            pltpu.VMEM((EMB, DIM), F32),
            pltpu.SemaphoreType.DMA((2,)),
            pltpu.SemaphoreType.DMA((2,)),
        ],
    )(input_data_first, input_data_last,
      Win, b_in.reshape(1, EMB), w1, c1b.reshape(1, EMB), w2,
      c2b.reshape(1, EMB), codebooks, rv, Wout, b_out.reshape(1, DIM))

    return out, p.reshape(())


# speedup vs baseline: 54.1161x; 1.0131x over previous
import jax
import jax.numpy as jnp
from jax import lax
from jax.experimental import pallas as pl
from jax.experimental.pallas import tpu as pltpu

B = 64
EMB = 256
DIM = 1024
K = 8192
F32 = jnp.float32

NBUF = 3


def _row_copies(xf_hbm, xl_hbm, xbuf, sem, r, slot):
    cps = []
    for c in range(8):
        p = 8 * r + c
        cps.append(pltpu.make_async_copy(
            xf_hbm.at[:, p, :], xbuf.at[slot, c], sem.at[slot]))
        cps.append(pltpu.make_async_copy(
            xl_hbm.at[:, p, :], xbuf.at[slot, 8 + c], sem.at[slot]))
    return cps


def _body(xf_hbm, xl_hbm, win_ref, bin_ref, w1_ref, c1b_ref, w2_ref, c2b_ref,
          cb_hbm, rv_ref, wout_hbm, bout_ref, out_ref, p_ref,
          proj_s, xbuf, e_s, cb_s, wout_s, sem, wsem):
    r = pl.program_id(0)
    cb_cp = pltpu.make_async_copy(cb_hbm, cb_s, wsem.at[0])
    wout_cp = pltpu.make_async_copy(wout_hbm, wout_s, wsem.at[1])

    @pl.when(r == 0)
    def _prime():
        cb_cp.start()
        wout_cp.start()
        for cp in _row_copies(xf_hbm, xl_hbm, xbuf, sem, 0, 0):
            cp.start()
        for cp in _row_copies(xf_hbm, xl_hbm, xbuf, sem, 1, 1):
            cp.start()

    @pl.when(r < 6)
    def _prefetch():
        for cp in _row_copies(xf_hbm, xl_hbm, xbuf, sem, r + 2, (r + 2) % NBUF):
            cp.start()

    slot = r % NBUF
    for cp in _row_copies(xf_hbm, xl_hbm, xbuf, sem, r, slot):
        cp.wait()

    xrow = xbuf[slot].reshape(16 * B, DIM)
    prow = jnp.dot(xrow, win_ref[...], preferred_element_type=F32) + bin_ref[...]
    for c in range(8):
        gs = (2 * (r % 2) + (c % 2)) * 16 + (r // 2) * 4 + (c // 2)
        proj_s[gs, pl.ds(0, B)] = prow[c * B:(c + 1) * B]
        proj_s[gs, pl.ds(B, B)] = prow[(8 + c) * B:(9 + c) * B]

    @pl.when(r % 2 == 1)
    def _conv_row():
        i = (r - 1) // 2
        z1 = jnp.zeros((1, 2 * B, EMB), F32)

        def shift_j(t):
            return jnp.concatenate([z1, t[0:3]], axis=0)

        acc = c1b_ref[...].astype(F32)
        for kh in range(3):
            for kw in range(3):
                a = 0 if kh == 1 else 1
                b = 0 if kw == 1 else 1
                g = a * 2 + b
                if kh == 0:
                    base = g * 16 + (i - 1) * 4
                    t = jnp.where(i > 0,
                                  proj_s[pl.ds(jnp.maximum(base, 0), 4)], 0.0)
                else:
                    t = proj_s[pl.ds(g * 16 + i * 4, 4)]
                if kw == 0:
                    t = shift_j(t)
                acc = acc + lax.dot_general(
                    t.reshape(4 * 2 * B, EMB), w1_ref[kh * 3 + kw],
                    (((1,), (1,)), ((), ())), preferred_element_type=F32)
        y = jnp.maximum(acc, 0.0)
        y3 = y.reshape(4, 2 * B, EMB)
        part = c2b_ref[...].astype(F32) * 0.25
        for j in range(4):
            part = part + lax.dot_general(
                y3[j], w2_ref[i, j], (((1,), (1,)), ((), ())),
                preferred_element_type=F32)

        @pl.when(r == 1)
        def _init():
            e_s[...] = part

        @pl.when(r > 1)
        def _accum():
            e_s[...] = e_s[...] + part

    @pl.when(r == 7)
    def _final():
        e = e_s[...]
        z = e[B:] - e[:B]

        cb_cp.wait()
        cb = cb_s[...]
        zc = lax.dot_general(z, cb, (((1,), (1,)), ((), ())),
                             preferred_element_type=F32)
        cc = lax.dot_general(jnp.ones((1, EMB), F32), cb * cb,
                             (((1,), (1,)), ((), ())),
                             preferred_element_type=F32)
        d = cc - 2.0 * zc
        dmin = jnp.min(d, axis=1, keepdims=True)

        zz = jnp.sum(z * z, axis=1, keepdims=True)
        nq = jnp.sqrt(jnp.maximum(dmin + zz, 0.0))
        rv = rv_ref[...]
        nr = jnp.sqrt(jnp.sum(rv * rv, axis=1, keepdims=True))
        q = z + (nq / (nr + 1e-12)) * rv
        wout_cp.wait()
        out = jnp.dot(q, wout_s[...], preferred_element_type=F32) + bout_ref[...]
        out_ref[...] = out.reshape(B, 1, DIM)

        m = (d <= dmin).astype(F32)
        colcnt = lax.dot_general(jnp.ones((1, B), F32), m,
                                 (((1,), (0,)), ((), ())),
                                 preferred_element_type=F32)
        cnt = lax.dot_general(m, colcnt, (((1,), (1,)), ((), ())),
                              preferred_element_type=F32)
        h = -jnp.sum(jnp.log(cnt * (1.0 / B) + 1e-10)) * (1.0 / B)
        p_ref[...] = jnp.exp(h).reshape(1, 1)


def kernel(input_data_first, input_data_last, codebooks, Win, b_in, Wout, b_out, c1w, c1b, c2w, c2b):
    w1 = jnp.transpose(c1w.reshape(EMB * EMB, 9)).reshape(9, EMB, EMB)
    w2 = jnp.transpose(c2w.reshape(EMB * EMB, 16)).reshape(4, 4, EMB, EMB)
    rv = jax.random.normal(jax.random.key(42), (B, EMB), dtype=F32)

    const = lambda shape: pl.BlockSpec(shape, lambda r: (0,) * len(shape))
    out, p = pl.pallas_call(
        _body,
        grid=(8,),
        in_specs=[
            pl.BlockSpec(memory_space=pl.ANY),
            pl.BlockSpec(memory_space=pl.ANY),
            const((DIM, EMB)),
            const((1, EMB)),
            const((9, EMB, EMB)),
            const((1, EMB)),
            const((4, 4, EMB, EMB)),
            const((1, EMB)),
            pl.BlockSpec(memory_space=pl.ANY),
            const((B, EMB)),
            pl.BlockSpec(memory_space=pl.ANY),
            const((1, DIM)),
        ],
        out_specs=[
            const((B, 1, DIM)),
            const((1, 1)),
        ],
        out_shape=[
            jax.ShapeDtypeStruct((B, 1, DIM), F32),
            jax.ShapeDtypeStruct((1, 1), F32),
        ],
        scratch_shapes=[
            pltpu.VMEM((64, 2 * B, EMB), F32),
            pltpu.VMEM((NBUF, 16, B, DIM), F32),
            pltpu.VMEM((2 * B, EMB), F32),
            pltpu.VMEM((K, EMB), F32),
            pltpu.VMEM((EMB, DIM), F32),
            pltpu.SemaphoreType.DMA((NBUF,)),
            pltpu.SemaphoreType.DMA((2,)),
        ],
    )(input_data_first, input_data_last,
      Win, b_in.reshape(1, EMB), w1, c1b.reshape(1, EMB), w2,
      c2b.reshape(1, EMB), codebooks, rv, Wout, b_out.reshape(1, DIM))

    return out, p.reshape(())


# speedup vs baseline: 61.5741x; 1.1378x over previous
import jax
import jax.numpy as jnp
from jax import lax
from jax.experimental import pallas as pl
from jax.experimental.pallas import tpu as pltpu

B = 64
EMB = 256
DIM = 1024
K = 8192
F32 = jnp.float32

NBUF = 3


def _row_copies(xf_hbm, xl_hbm, xbuf, sem, r, slot):
    cps = []
    for c in range(8):
        p = 8 * r + c
        cps.append(pltpu.make_async_copy(
            xf_hbm.at[:, p, :], xbuf.at[slot, c], sem.at[slot]))
        cps.append(pltpu.make_async_copy(
            xl_hbm.at[:, p, :], xbuf.at[slot, 8 + c], sem.at[slot]))
    return cps


def _body(xf_hbm, xl_hbm, win_ref, bin_ref, w1_ref, c1b_ref, w2_ref, c2b_ref,
          cb_hbm, rv_ref, wout_hbm, bout_ref, out_ref, p_ref,
          proj_s, xbuf, e_s, cb_s, wout_s, cc_s, sem, wsem):
    r = pl.program_id(0)
    cb_cp = pltpu.make_async_copy(cb_hbm, cb_s, wsem.at[0])
    wout_cp = pltpu.make_async_copy(wout_hbm, wout_s, wsem.at[1])

    @pl.when(r == 0)
    def _prime():
        for cp in _row_copies(xf_hbm, xl_hbm, xbuf, sem, 0, 0):
            cp.start()
        for cp in _row_copies(xf_hbm, xl_hbm, xbuf, sem, 1, 1):
            cp.start()

    @pl.when(r < 6)
    def _prefetch():
        for cp in _row_copies(xf_hbm, xl_hbm, xbuf, sem, r + 2, (r + 2) % NBUF):
            cp.start()

    @pl.when(r == 2)
    def _fetch_cb():
        cb_cp.start()

    @pl.when(r == 5)
    def _fetch_wout():
        wout_cp.start()

    slot = r % NBUF
    for cp in _row_copies(xf_hbm, xl_hbm, xbuf, sem, r, slot):
        cp.wait()

    xrow = xbuf[slot].reshape(16 * B, DIM)
    prow = jnp.dot(xrow, win_ref[...], preferred_element_type=F32) + bin_ref[...]
    for c in range(8):
        gs = (2 * (r % 2) + (c % 2)) * 16 + (r // 2) * 4 + (c // 2)
        proj_s[gs, pl.ds(0, B)] = prow[c * B:(c + 1) * B]
        proj_s[gs, pl.ds(B, B)] = prow[(8 + c) * B:(9 + c) * B]

    @pl.when(r % 2 == 1)
    def _conv_row():
        i = (r - 1) // 2
        z1 = jnp.zeros((1, 2 * B, EMB), F32)

        def shift_j(t):
            return jnp.concatenate([z1, t[0:3]], axis=0)

        acc = c1b_ref[...].astype(F32)
        for kh in range(3):
            for kw in range(3):
                a = 0 if kh == 1 else 1
                b = 0 if kw == 1 else 1
                g = a * 2 + b
                if kh == 0:
                    base = g * 16 + (i - 1) * 4
                    t = jnp.where(i > 0,
                                  proj_s[pl.ds(jnp.maximum(base, 0), 4)], 0.0)
                else:
                    t = proj_s[pl.ds(g * 16 + i * 4, 4)]
                if kw == 0:
                    t = shift_j(t)
                acc = acc + lax.dot_general(
                    t.reshape(4 * 2 * B, EMB), w1_ref[kh * 3 + kw],
                    (((1,), (1,)), ((), ())), preferred_element_type=F32)
        y = jnp.maximum(acc, 0.0)
        y3 = y.reshape(4, 2 * B, EMB)
        part = c2b_ref[...].astype(F32) * 0.25
        for j in range(4):
            part = part + lax.dot_general(
                y3[j], w2_ref[i, j], (((1,), (1,)), ((), ())),
                preferred_element_type=F32)

        @pl.when(r == 1)
        def _init():
            e_s[...] = part

        @pl.when(r > 1)
        def _accum():
            e_s[...] = e_s[...] + part

    @pl.when(r == 6)
    def _codebook_norms():
        cb_cp.wait()
        cb = cb_s[...]
        cc_s[...] = lax.dot_general(jnp.ones((1, EMB), F32), cb * cb,
                                    (((1,), (1,)), ((), ())),
                                    preferred_element_type=F32)

    @pl.when(r == 7)
    def _final():
        e = e_s[...]
        z = e[B:] - e[:B]

        zc = lax.dot_general(z, cb_s[...], (((1,), (1,)), ((), ())),
                             preferred_element_type=F32)
        d = cc_s[...] - 2.0 * zc
        dmin = jnp.min(d, axis=1, keepdims=True)

        zz = jnp.sum(z * z, axis=1, keepdims=True)
        nq = jnp.sqrt(jnp.maximum(dmin + zz, 0.0))
        rv = rv_ref[...]
        nr = jnp.sqrt(jnp.sum(rv * rv, axis=1, keepdims=True))
        q = z + (nq / (nr + 1e-12)) * rv
        wout_cp.wait()
        out = jnp.dot(q, wout_s[...], preferred_element_type=F32) + bout_ref[...]
        out_ref[...] = out.reshape(B, 1, DIM)

        m = (d <= dmin).astype(F32)
        colcnt = lax.dot_general(jnp.ones((1, B), F32), m,
                                 (((1,), (0,)), ((), ())),
                                 preferred_element_type=F32)
        cnt = lax.dot_general(m, colcnt, (((1,), (1,)), ((), ())),
                              preferred_element_type=F32)
        h = -jnp.sum(jnp.log(cnt * (1.0 / B) + 1e-10)) * (1.0 / B)
        p_ref[...] = jnp.exp(h).reshape(1, 1)


def kernel(input_data_first, input_data_last, codebooks, Win, b_in, Wout, b_out, c1w, c1b, c2w, c2b):
    w1 = jnp.transpose(c1w.reshape(EMB, EMB, 9), (2, 0, 1))
    w2 = jnp.transpose(c2w.reshape(EMB, EMB, 16), (2, 0, 1)).reshape(4, 4, EMB, EMB)
    rv = jax.random.normal(jax.random.key(42), (B, EMB), dtype=F32)

    const = lambda shape: pl.BlockSpec(shape, lambda r: (0,) * len(shape))
    out, p = pl.pallas_call(
        _body,
        grid=(8,),
        in_specs=[
            pl.BlockSpec(memory_space=pl.ANY),
            pl.BlockSpec(memory_space=pl.ANY),
            const((DIM, EMB)),
            const((1, EMB)),
            const((9, EMB, EMB)),
            const((1, EMB)),
            const((4, 4, EMB, EMB)),
            const((1, EMB)),
            pl.BlockSpec(memory_space=pl.ANY),
            const((B, EMB)),
            pl.BlockSpec(memory_space=pl.ANY),
            const((1, DIM)),
        ],
        out_specs=[
            const((B, 1, DIM)),
            const((1, 1)),
        ],
        out_shape=[
            jax.ShapeDtypeStruct((B, 1, DIM), F32),
            jax.ShapeDtypeStruct((1, 1), F32),
        ],
        scratch_shapes=[
            pltpu.VMEM((64, 2 * B, EMB), F32),
            pltpu.VMEM((NBUF, 16, B, DIM), F32),
            pltpu.VMEM((2 * B, EMB), F32),
            pltpu.VMEM((K, EMB), F32),
            pltpu.VMEM((EMB, DIM), F32),
            pltpu.VMEM((1, K), F32),
            pltpu.SemaphoreType.DMA((NBUF,)),
            pltpu.SemaphoreType.DMA((2,)),
        ],
    )(input_data_first, input_data_last,
      Win, b_in.reshape(1, EMB), w1, c1b.reshape(1, EMB), w2,
      c2b.reshape(1, EMB), codebooks, rv, Wout, b_out.reshape(1, DIM))

    return out, p.reshape(())
